# Initial kernel scaffold; baseline (speedup 1.0000x reference)
import jax
import jax.numpy as jnp
from jax.experimental import pallas as pl


def kernel(x, edge_index, W1, b1, W2, b2):
    raise NotImplementedError("write your pallas kernel here")


# speedup vs baseline: 5.0425x; 5.0425x over previous
import functools

import jax
import jax.numpy as jnp
from jax import lax
from jax.experimental import pallas as pl
from jax.experimental.pallas import tpu as pltpu
from jax.experimental.pallas import tpu_sc as plsc

N = 10000
E = 320000
NC = 2
NS = 16
NW = NC * NS
EPT = E // NW
CH = 80
NCHUNK = EPT // CH
NP = 10240
RPT = NP // NS


def _mesh():
    return plsc.VectorSubcoreMesh(core_axis_name="c", subcore_axis_name="s")


def _zero_fill(ref, nrow, ncol):
    z = jnp.zeros((16,), jnp.float32)

    def body(i, carry):
        r = i // (ncol // 16)
        c = i % (ncol // 16)
        ref[r, pl.ds(c * 16, 16)] = z
        return carry

    lax.fori_loop(0, nrow * (ncol // 16), body, 0)


def _one_fill(ref, nrow, ncol):
    o = jnp.ones((16,), jnp.float32)

    def body(i, carry):
        r = i // (ncol // 16)
        c = i % (ncol // 16)
        ref[r, pl.ds(c * 16, 16)] = o
        return carry

    lax.fori_loop(0, nrow * (ncol // 16), body, 0)


@functools.partial(
    pl.kernel,
    out_type=(
        jax.ShapeDtypeStruct((NC, NP, 16), jnp.float32),
        jax.ShapeDtypeStruct((NC, NP, 16), jnp.float32),
    ),
    mesh=_mesh(),
    compiler_params=pltpu.CompilerParams(use_tc_tiling_on_sc=False),
    scratch_types=[
        pltpu.VMEM((CH,), jnp.int32),
        pltpu.VMEM((CH,), jnp.int32),
        pltpu.VMEM((CH, 16), jnp.float32),
        pltpu.VMEM((CH, 16), jnp.float32),
        pltpu.VMEM_SHARED((NP, 16), jnp.float32),
        pltpu.VMEM_SHARED((NP, 16), jnp.float32),
    ],
)
def _sc_degrees(src_hbm, dst_hbm, do_hbm, di_hbm, idx_s, idx_d, ones_v,
                z_v, do_sh, di_sh):
    cid = lax.axis_index("c")
    sid = lax.axis_index("s")
    w = cid * NS + sid

    _one_fill(ones_v, CH, 16)
    _zero_fill(z_v, CH, 16)

    def zb(i, carry):
        r0 = sid * RPT + i * CH
        pltpu.sync_copy(z_v, do_sh.at[pl.ds(r0, CH)])
        pltpu.sync_copy(z_v, di_sh.at[pl.ds(r0, CH)])
        return carry

    lax.fori_loop(0, RPT // CH, zb, 0)
    plsc.subcore_barrier()

    def body(c, carry):
        base = w * EPT + c * CH
        pltpu.sync_copy(src_hbm.at[pl.ds(base, CH)], idx_s)
        pltpu.sync_copy(dst_hbm.at[pl.ds(base, CH)], idx_d)
        pltpu.sync_copy(ones_v, do_sh.at[idx_s], add=True)
        pltpu.sync_copy(ones_v, di_sh.at[idx_d], add=True)
        return carry

    lax.fori_loop(0, NCHUNK, body, 0)
    plsc.subcore_barrier()

    r0 = sid * RPT
    pltpu.sync_copy(do_sh.at[pl.ds(r0, RPT)], do_hbm.at[cid, pl.ds(r0, RPT)])
    pltpu.sync_copy(di_sh.at[pl.ds(r0, RPT)], di_hbm.at[cid, pl.ds(r0, RPT)])


def _make_sc_agg(D):
    @functools.partial(
        pl.kernel,
        out_type=jax.ShapeDtypeStruct((NC, NP, D), jnp.float32),
        mesh=_mesh(),
        compiler_params=pltpu.CompilerParams(use_tc_tiling_on_sc=(D == 128)),
        scratch_types=[
            pltpu.VMEM((CH,), jnp.int32),
            pltpu.VMEM((CH,), jnp.int32),
            pltpu.VMEM((CH, D), jnp.float32),
            pltpu.VMEM_SHARED((NP, D), jnp.float32),
            pltpu.SemaphoreType.DMA,
        ],
    )
    def _sc_agg(y_hbm, src_hbm, dst_hbm, out_hbm, idx_s, idx_d, rows,
                agg_sh, sem):
        cid = lax.axis_index("c")
        sid = lax.axis_index("s")
        w = cid * NS + sid

        _zero_fill(rows, CH, D)

        def zb(i, carry):
            pltpu.sync_copy(rows, agg_sh.at[pl.ds(sid * RPT + i * CH, CH)])
            return carry

        lax.fori_loop(0, RPT // CH, zb, 0)
        plsc.subcore_barrier()

        def body(c, carry):
            base = w * EPT + c * CH
            pltpu.sync_copy(src_hbm.at[pl.ds(base, CH)], idx_s)
            pltpu.sync_copy(dst_hbm.at[pl.ds(base, CH)], idx_d)
            pltpu.async_copy(y_hbm.at[idx_s], rows, sem).wait()
            pltpu.sync_copy(rows, agg_sh.at[idx_d], add=True)
            return carry

        lax.fori_loop(0, NCHUNK, body, 0)
        plsc.subcore_barrier()

        r0 = sid * RPT
        pltpu.sync_copy(agg_sh.at[pl.ds(r0, RPT)],
                        out_hbm.at[cid, pl.ds(r0, RPT)])

    return _sc_agg


_sc_agg128 = _make_sc_agg(128)
_sc_agg64 = _make_sc_agg(64)


def _tc_prep_body(do_ref, di_ref, x_ref, norm_ref, y_ref):
    deg_o = do_ref[0] + do_ref[1]
    deg_i = di_ref[0] + di_ref[1]
    ns = lax.rsqrt(jnp.maximum(deg_o, 1.0))
    nd = lax.rsqrt(jnp.maximum(deg_i, 1.0))
    norm_ref[...] = jnp.concatenate([ns, nd], axis=1)
    y_ref[...] = x_ref[...] * ns[:N, 0:1]


def _tc_prep(do_p, di_p, x):
    return pl.pallas_call(
        _tc_prep_body,
        out_shape=(
            jax.ShapeDtypeStruct((NP, 32), jnp.float32),
            jax.ShapeDtypeStruct((N, 128), jnp.float32),
        ),
    )(do_p, di_p, x)


def _tc_layer1_body(p_ref, norm_ref, w1_ref, b1_ref, w2_ref, y2_ref):
    agg = (p_ref[0] + p_ref[1]) * norm_ref[:, 16:17]
    h = jnp.maximum(
        jnp.dot(agg, w1_ref[...], preferred_element_type=jnp.float32)
        + b1_ref[...], 0.0)
    y2_ref[...] = jnp.dot(h, w2_ref[...],
                          preferred_element_type=jnp.float32) * norm_ref[:, 0:1]


def _tc_layer1(part1, norm, W1, b1, W2):
    blk = 1000
    return pl.pallas_call(
        _tc_layer1_body,
        grid=(N // blk,),
        in_specs=[
            pl.BlockSpec((2, blk, 128), lambda i: (0, i, 0)),
            pl.BlockSpec((blk, 32), lambda i: (i, 0)),
            pl.BlockSpec((128, 128), lambda i: (0, 0)),
            pl.BlockSpec((1, 128), lambda i: (0, 0)),
            pl.BlockSpec((128, 64), lambda i: (0, 0)),
        ],
        out_specs=pl.BlockSpec((blk, 64), lambda i: (i, 0)),
        out_shape=jax.ShapeDtypeStruct((N, 64), jnp.float32),
    )(part1, norm, W1, b1, W2)


def _tc_out_body(p_ref, norm_ref, b2_ref, out_ref):
    out_ref[...] = ((p_ref[0] + p_ref[1]) * norm_ref[:, 16:17]
                    + b2_ref[...])


def _tc_out(part2, norm, b2):
    blk = 1000
    return pl.pallas_call(
        _tc_out_body,
        grid=(N // blk,),
        in_specs=[
            pl.BlockSpec((2, blk, 64), lambda i: (0, i, 0)),
            pl.BlockSpec((blk, 32), lambda i: (i, 0)),
            pl.BlockSpec((1, 64), lambda i: (0, 0)),
        ],
        out_specs=pl.BlockSpec((blk, 64), lambda i: (i, 0)),
        out_shape=jax.ShapeDtypeStruct((N, 64), jnp.float32),
    )(part2, norm, b2)


def kernel(x, edge_index, W1, b1, W2, b2):
    src = edge_index[0]
    dst = edge_index[1]
    do_p, di_p = _sc_degrees(src, dst)
    norm, y1 = _tc_prep(do_p, di_p, x)
    part1 = _sc_agg128(y1, src, dst)
    y2 = _tc_layer1(part1, norm, W1, b1.reshape(1, -1), W2)
    part2 = _sc_agg64(y2, src, dst)
    return _tc_out(part2, norm, b2.reshape(1, -1))


# speedup vs baseline: 10.9764x; 2.1768x over previous
import functools

import jax
import jax.numpy as jnp
from jax import lax
from jax.experimental import pallas as pl
from jax.experimental.pallas import tpu as pltpu
from jax.experimental.pallas import tpu_sc as plsc

N = 10000
E = 320000
NC = 2
NS = 16
NW = NC * NS
EPT = E // NW
CH = 100
NCH = EPT // CH
NP = 10240
RPT = NP // NS
ZR = 64


def _mesh():
    return plsc.VectorSubcoreMesh(core_axis_name="c", subcore_axis_name="s")


def _fill(ref, nrow, ncol, val):
    v = jnp.full((16,), val, jnp.float32)

    def body(i, carry):
        r = i // (ncol // 16)
        c = i % (ncol // 16)
        ref[r, pl.ds(c * 16, 16)] = v
        return carry

    lax.fori_loop(0, nrow * (ncol // 16), body, 0)


@functools.partial(
    pl.kernel,
    out_type=(
        jax.ShapeDtypeStruct((NC, NP, 16), jnp.float32),
        jax.ShapeDtypeStruct((NC, NP, 16), jnp.float32),
    ),
    mesh=_mesh(),
    compiler_params=pltpu.CompilerParams(use_tc_tiling_on_sc=False),
    scratch_types=[
        pltpu.VMEM((NCH, CH), jnp.int32),
        pltpu.VMEM((NCH, CH), jnp.int32),
        pltpu.VMEM((CH, 16), jnp.float32),
        pltpu.VMEM((ZR, 16), jnp.float32),
        pltpu.VMEM_SHARED((NP, 16), jnp.float32),
        pltpu.VMEM_SHARED((NP, 16), jnp.float32),
        pltpu.SemaphoreType.DMA,
        pltpu.SemaphoreType.DMA,
    ],
)
def _sc_degrees(src_hbm, dst_hbm, do_hbm, di_hbm, isall, idall, ones_v,
                z_v, do_sh, di_sh, semA, semB):
    cid = lax.axis_index("c")
    sid = lax.axis_index("s")
    w = cid * NS + sid

    pltpu.sync_copy(src_hbm.at[w], isall)
    pltpu.sync_copy(dst_hbm.at[w], idall)
    _fill(ones_v, CH, 16, 1.0)
    _fill(z_v, ZR, 16, 0.0)

    def zb(i, carry):
        r0 = sid * RPT + i * ZR
        pltpu.sync_copy(z_v, do_sh.at[pl.ds(r0, ZR)])
        pltpu.sync_copy(z_v, di_sh.at[pl.ds(r0, ZR)])
        return carry

    lax.fori_loop(0, RPT // ZR, zb, 0)
    plsc.subcore_barrier()

    def body(i, carry):
        pltpu.async_copy(ones_v, do_sh.at[isall.at[i]], semA, add=True)
        pltpu.async_copy(ones_v, di_sh.at[idall.at[i]], semB, add=True)

        @pl.when(i > 0)
        def _():
            pltpu.make_async_copy(ones_v, do_sh.at[isall.at[i - 1]],
                                  semA).wait()
            pltpu.make_async_copy(ones_v, di_sh.at[idall.at[i - 1]],
                                  semB).wait()

        return carry

    lax.fori_loop(0, NCH, body, 0)
    pltpu.make_async_copy(ones_v, do_sh.at[isall.at[NCH - 1]], semA).wait()
    pltpu.make_async_copy(ones_v, di_sh.at[idall.at[NCH - 1]], semB).wait()
    plsc.subcore_barrier()

    r0 = sid * RPT
    pltpu.sync_copy(do_sh.at[pl.ds(r0, RPT)], do_hbm.at[cid, pl.ds(r0, RPT)])
    pltpu.sync_copy(di_sh.at[pl.ds(r0, RPT)], di_hbm.at[cid, pl.ds(r0, RPT)])


def _make_sc_agg(D):
    @functools.partial(
        pl.kernel,
        out_type=jax.ShapeDtypeStruct((NC, NP, D), jnp.float32),
        mesh=_mesh(),
        compiler_params=pltpu.CompilerParams(use_tc_tiling_on_sc=False),
        scratch_types=[
            pltpu.VMEM((NCH, CH), jnp.int32),
            pltpu.VMEM((NCH, CH), jnp.int32),
            pltpu.VMEM((CH, D), jnp.float32),
            pltpu.VMEM((CH, D), jnp.float32),
            pltpu.VMEM_SHARED((NP, D), jnp.float32),
            pltpu.SemaphoreType.DMA,
            pltpu.SemaphoreType.DMA,
            pltpu.SemaphoreType.DMA,
            pltpu.SemaphoreType.DMA,
        ],
    )
    def _sc_agg(y_hbm, src_hbm, dst_hbm, out_hbm, isall, idall, rows0, rows1,
                agg_sh, semg0, semg1, sems0, sems1):
        cid = lax.axis_index("c")
        sid = lax.axis_index("s")
        w = cid * NS + sid

        pltpu.sync_copy(src_hbm.at[w], isall)
        pltpu.sync_copy(dst_hbm.at[w], idall)

        _fill(rows0, CH, D, 0.0)

        def zb(i, carry):
            pltpu.sync_copy(rows0, agg_sh.at[pl.ds(sid * RPT + i * CH, CH)])
            return carry

        lax.fori_loop(0, RPT // CH, zb, 0)
        pltpu.sync_copy(rows0.at[pl.ds(0, RPT % CH)],
                        agg_sh.at[pl.ds(sid * RPT + (RPT // CH) * CH,
                                        RPT % CH)])
        plsc.subcore_barrier()

        pltpu.async_copy(y_hbm.at[isall.at[0]], rows0, semg0)
        pltpu.async_copy(y_hbm.at[isall.at[1]], rows1, semg1)

        def body(i, carry):
            cc = 2 * i
            pltpu.make_async_copy(y_hbm.at[isall.at[cc]], rows0, semg0).wait()
            pltpu.async_copy(rows0, agg_sh.at[idall.at[cc]], sems0, add=True)
            pltpu.make_async_copy(y_hbm.at[isall.at[cc + 1]], rows1,
                                  semg1).wait()
            pltpu.async_copy(rows1, agg_sh.at[idall.at[cc + 1]], sems1,
                             add=True)
            pltpu.make_async_copy(rows0, agg_sh.at[idall.at[cc]],
                                  sems0).wait()

            @pl.when(cc + 2 < NCH)
            def _():
                pltpu.async_copy(y_hbm.at[isall.at[cc + 2]], rows0, semg0)

            pltpu.make_async_copy(rows1, agg_sh.at[idall.at[cc + 1]],
                                  sems1).wait()

            @pl.when(cc + 3 < NCH)
            def _():
                pltpu.async_copy(y_hbm.at[isall.at[cc + 3]], rows1, semg1)

            return carry

        lax.fori_loop(0, NCH // 2, body, 0)
        plsc.subcore_barrier()

        r0 = sid * RPT
        pltpu.sync_copy(agg_sh.at[pl.ds(r0, RPT)],
                        out_hbm.at[cid, pl.ds(r0, RPT)])

    return _sc_agg


_sc_agg128 = _make_sc_agg(128)
_sc_agg64 = _make_sc_agg(64)


def _tc_prep_body(do_ref, di_ref, x_ref, norm_ref, y_ref):
    deg_o = do_ref[0] + do_ref[1]
    deg_i = di_ref[0] + di_ref[1]
    ns = lax.rsqrt(jnp.maximum(deg_o, 1.0))
    nd = lax.rsqrt(jnp.maximum(deg_i, 1.0))
    norm_ref[...] = jnp.concatenate([ns, nd], axis=1)
    y_ref[...] = x_ref[...] * ns[:N, 0:1]


def _tc_prep(do_p, di_p, x):
    return pl.pallas_call(
        _tc_prep_body,
        out_shape=(
            jax.ShapeDtypeStruct((NP, 32), jnp.float32),
            jax.ShapeDtypeStruct((N, 128), jnp.float32),
        ),
    )(do_p, di_p, x)


def _tc_layer1_body(p_ref, norm_ref, w1_ref, b1_ref, w2_ref, y2_ref):
    agg = (p_ref[0] + p_ref[1]) * norm_ref[:, 16:17]
    h = jnp.maximum(
        jnp.dot(agg, w1_ref[...], preferred_element_type=jnp.float32)
        + b1_ref[...], 0.0)
    y2_ref[...] = jnp.dot(h, w2_ref[...],
                          preferred_element_type=jnp.float32) * norm_ref[:, 0:1]


def _tc_layer1(part1, norm, W1, b1, W2):
    blk = 1000
    return pl.pallas_call(
        _tc_layer1_body,
        grid=(N // blk,),
        in_specs=[
            pl.BlockSpec((2, blk, 128), lambda i: (0, i, 0)),
            pl.BlockSpec((blk, 32), lambda i: (i, 0)),
            pl.BlockSpec((128, 128), lambda i: (0, 0)),
            pl.BlockSpec((1, 128), lambda i: (0, 0)),
            pl.BlockSpec((128, 64), lambda i: (0, 0)),
        ],
        out_specs=pl.BlockSpec((blk, 64), lambda i: (i, 0)),
        out_shape=jax.ShapeDtypeStruct((N, 64), jnp.float32),
    )(part1, norm, W1, b1, W2)


def _tc_out_body(p_ref, norm_ref, b2_ref, out_ref):
    out_ref[...] = ((p_ref[0] + p_ref[1]) * norm_ref[:, 16:17]
                    + b2_ref[...])


def _tc_out(part2, norm, b2):
    blk = 1000
    return pl.pallas_call(
        _tc_out_body,
        grid=(N // blk,),
        in_specs=[
            pl.BlockSpec((2, blk, 64), lambda i: (0, i, 0)),
            pl.BlockSpec((blk, 32), lambda i: (i, 0)),
            pl.BlockSpec((1, 64), lambda i: (0, 0)),
        ],
        out_specs=pl.BlockSpec((blk, 64), lambda i: (i, 0)),
        out_shape=jax.ShapeDtypeStruct((N, 64), jnp.float32),
    )(part2, norm, b2)


def kernel(x, edge_index, W1, b1, W2, b2):
    src = edge_index[0].reshape(NW, NCH, CH)
    dst = edge_index[1].reshape(NW, NCH, CH)
    do_p, di_p = _sc_degrees(src, dst)
    norm, y1 = _tc_prep(do_p, di_p, x)
    part1 = _sc_agg128(y1, src, dst)
    y2 = _tc_layer1(part1, norm, W1, b1.reshape(1, -1), W2)
    part2 = _sc_agg64(y2, src, dst)
    return _tc_out(part2, norm, b2.reshape(1, -1))


# speedup vs baseline: 12.2750x; 1.1183x over previous
import functools

import jax
import jax.numpy as jnp
from jax import lax
from jax.experimental import pallas as pl
from jax.experimental.pallas import tpu as pltpu
from jax.experimental.pallas import tpu_sc as plsc

N = 10000
E = 320000
NC = 2
NS = 16
NW = NC * NS
EPT = E // NW
CH = 100
NCH = EPT // CH
NP = N
RPT = NP // NS
ZR = 125


def _mesh():
    return plsc.VectorSubcoreMesh(core_axis_name="c", subcore_axis_name="s")


def _fill(ref, nrow, ncol, val):
    v = jnp.full((16,), val, jnp.float32)

    def body(i, carry):
        r = i // (ncol // 16)
        c = i % (ncol // 16)
        ref[r, pl.ds(c * 16, 16)] = v
        return carry

    lax.fori_loop(0, nrow * (ncol // 16), body, 0)


@functools.partial(
    pl.kernel,
    out_type=(
        jax.ShapeDtypeStruct((NC, NP, 16), jnp.float32),
        jax.ShapeDtypeStruct((NC, NP, 16), jnp.float32),
    ),
    mesh=_mesh(),
    compiler_params=pltpu.CompilerParams(use_tc_tiling_on_sc=False),
    scratch_types=[
        pltpu.VMEM((NCH, CH), jnp.int32),
        pltpu.VMEM((NCH, CH), jnp.int32),
        pltpu.VMEM((CH, 16), jnp.float32),
        pltpu.VMEM((ZR, 16), jnp.float32),
        pltpu.VMEM_SHARED((NP, 16), jnp.float32),
        pltpu.VMEM_SHARED((NP, 16), jnp.float32),
        pltpu.SemaphoreType.DMA,
        pltpu.SemaphoreType.DMA,
    ],
)
def _sc_degrees(src_hbm, dst_hbm, do_hbm, di_hbm, isall, idall, ones_v,
                z_v, do_sh, di_sh, semA, semB):
    cid = lax.axis_index("c")
    sid = lax.axis_index("s")
    w = cid * NS + sid

    pltpu.sync_copy(src_hbm.at[w], isall)
    pltpu.sync_copy(dst_hbm.at[w], idall)
    _fill(ones_v, CH, 16, 1.0)
    _fill(z_v, ZR, 16, 0.0)

    def zb(i, carry):
        r0 = sid * RPT + i * ZR
        pltpu.sync_copy(z_v, do_sh.at[pl.ds(r0, ZR)])
        pltpu.sync_copy(z_v, di_sh.at[pl.ds(r0, ZR)])
        return carry

    lax.fori_loop(0, RPT // ZR, zb, 0)
    plsc.subcore_barrier()

    def body(i, carry):
        pltpu.async_copy(ones_v, do_sh.at[isall.at[i]], semA, add=True)
        pltpu.async_copy(ones_v, di_sh.at[idall.at[i]], semB, add=True)

        @pl.when(i > 0)
        def _():
            pltpu.make_async_copy(ones_v, do_sh.at[isall.at[i - 1]],
                                  semA).wait()
            pltpu.make_async_copy(ones_v, di_sh.at[idall.at[i - 1]],
                                  semB).wait()

        return carry

    lax.fori_loop(0, NCH, body, 0)
    pltpu.make_async_copy(ones_v, do_sh.at[isall.at[NCH - 1]], semA).wait()
    pltpu.make_async_copy(ones_v, di_sh.at[idall.at[NCH - 1]], semB).wait()
    plsc.subcore_barrier()

    r0 = sid * RPT
    pltpu.sync_copy(do_sh.at[pl.ds(r0, RPT)], do_hbm.at[cid, pl.ds(r0, RPT)])
    pltpu.sync_copy(di_sh.at[pl.ds(r0, RPT)], di_hbm.at[cid, pl.ds(r0, RPT)])


def _make_sc_agg(D, NBUF):
    assert NCH % NBUF == 0

    @functools.partial(
        pl.kernel,
        out_type=jax.ShapeDtypeStruct((NC, NP, D), jnp.float32),
        mesh=_mesh(),
        compiler_params=pltpu.CompilerParams(use_tc_tiling_on_sc=False),
        scratch_types=[
            pltpu.VMEM((NCH, CH), jnp.int32),
            pltpu.VMEM((NCH, CH), jnp.int32),
            [pltpu.VMEM((CH, D), jnp.float32)] * NBUF,
            [pltpu.SemaphoreType.DMA] * NBUF,
            [pltpu.SemaphoreType.DMA] * NBUF,
            pltpu.VMEM_SHARED((NP, D), jnp.float32),
        ],
    )
    def _sc_agg(y_hbm, src_hbm, dst_hbm, out_hbm, isall, idall, rows,
                semg, sems, agg_sh):
        cid = lax.axis_index("c")
        sid = lax.axis_index("s")
        w = cid * NS + sid

        pltpu.sync_copy(src_hbm.at[w], isall)
        pltpu.sync_copy(dst_hbm.at[w], idall)

        _fill(rows[0], CH, D, 0.0)

        def zb(i, carry):
            pltpu.sync_copy(rows[0], agg_sh.at[pl.ds(sid * RPT + i * CH, CH)])
            return carry

        lax.fori_loop(0, RPT // CH, zb, 0)
        pltpu.sync_copy(rows[0].at[pl.ds(0, RPT % CH)],
                        agg_sh.at[pl.ds(sid * RPT + (RPT // CH) * CH,
                                        RPT % CH)])
        plsc.subcore_barrier()

        for b in range(NBUF):
            pltpu.async_copy(y_hbm.at[isall.at[b]], rows[b], semg[b])

        def body(g, carry):
            base = g * NBUF
            for b in range(NBUF):
                cc = base + b
                pltpu.make_async_copy(y_hbm.at[isall.at[cc]], rows[b],
                                      semg[b]).wait()
                pltpu.async_copy(rows[b], agg_sh.at[idall.at[cc]], sems[b],
                                 add=True)
                pb = (b - 1) % NBUF
                pcc = cc - 1

                def _drain_refill():
                    pltpu.make_async_copy(rows[pb], agg_sh.at[idall.at[pcc]],
                                          sems[pb]).wait()

                    @pl.when(pcc + NBUF < NCH)
                    def _():
                        pltpu.async_copy(y_hbm.at[isall.at[pcc + NBUF]],
                                         rows[pb], semg[pb])

                if b == 0:
                    pl.when(g > 0)(_drain_refill)
                else:
                    _drain_refill()

            return carry

        lax.fori_loop(0, NCH // NBUF, body, 0)
        pltpu.make_async_copy(rows[NBUF - 1], agg_sh.at[idall.at[NCH - 1]],
                              sems[NBUF - 1]).wait()
        plsc.subcore_barrier()

        r0 = sid * RPT
        pltpu.sync_copy(agg_sh.at[pl.ds(r0, RPT)],
                        out_hbm.at[cid, pl.ds(r0, RPT)])

    return _sc_agg


_sc_agg128 = _make_sc_agg(128, 2)
_sc_agg64 = _make_sc_agg(64, 4)


def _tc_prep_body(do_ref, di_ref, x_ref, norm_ref, y_ref):
    deg_o = do_ref[0] + do_ref[1]
    deg_i = di_ref[0] + di_ref[1]
    ns = lax.rsqrt(jnp.maximum(deg_o, 1.0))
    nd = lax.rsqrt(jnp.maximum(deg_i, 1.0))
    norm_ref[...] = jnp.concatenate([ns, nd], axis=1)
    y_ref[...] = x_ref[...] * ns[:N, 0:1]


def _tc_prep(do_p, di_p, x):
    return pl.pallas_call(
        _tc_prep_body,
        out_shape=(
            jax.ShapeDtypeStruct((NP, 32), jnp.float32),
            jax.ShapeDtypeStruct((N, 128), jnp.float32),
        ),
    )(do_p, di_p, x)


def _tc_layer1_body(p_ref, norm_ref, w1_ref, b1_ref, w2_ref, y2_ref):
    agg = (p_ref[0] + p_ref[1]) * norm_ref[:, 16:17]
    h = jnp.maximum(
        jnp.dot(agg, w1_ref[...], preferred_element_type=jnp.float32)
        + b1_ref[...], 0.0)
    y2_ref[...] = jnp.dot(h, w2_ref[...],
                          preferred_element_type=jnp.float32) * norm_ref[:, 0:1]


def _tc_layer1(part1, norm, W1, b1, W2):
    blk = 1000
    return pl.pallas_call(
        _tc_layer1_body,
        grid=(N // blk,),
        in_specs=[
            pl.BlockSpec((2, blk, 128), lambda i: (0, i, 0)),
            pl.BlockSpec((blk, 32), lambda i: (i, 0)),
            pl.BlockSpec((128, 128), lambda i: (0, 0)),
            pl.BlockSpec((1, 128), lambda i: (0, 0)),
            pl.BlockSpec((128, 64), lambda i: (0, 0)),
        ],
        out_specs=pl.BlockSpec((blk, 64), lambda i: (i, 0)),
        out_shape=jax.ShapeDtypeStruct((N, 64), jnp.float32),
    )(part1, norm, W1, b1, W2)


def _tc_out_body(p_ref, norm_ref, b2_ref, out_ref):
    out_ref[...] = ((p_ref[0] + p_ref[1]) * norm_ref[:, 16:17]
                    + b2_ref[...])


def _tc_out(part2, norm, b2):
    blk = 1000
    return pl.pallas_call(
        _tc_out_body,
        grid=(N // blk,),
        in_specs=[
            pl.BlockSpec((2, blk, 64), lambda i: (0, i, 0)),
            pl.BlockSpec((blk, 32), lambda i: (i, 0)),
            pl.BlockSpec((1, 64), lambda i: (0, 0)),
        ],
        out_specs=pl.BlockSpec((blk, 64), lambda i: (i, 0)),
        out_shape=jax.ShapeDtypeStruct((N, 64), jnp.float32),
    )(part2, norm, b2)


def kernel(x, edge_index, W1, b1, W2, b2):
    src = edge_index[0].reshape(NW, NCH, CH)
    dst = edge_index[1].reshape(NW, NCH, CH)
    do_p, di_p = _sc_degrees(src, dst)
    norm, y1 = _tc_prep(do_p, di_p, x)
    part1 = _sc_agg128(y1, src, dst)
    y2 = _tc_layer1(part1, norm, W1, b1.reshape(1, -1), W2)
    part2 = _sc_agg64(y2, src, dst)
    return _tc_out(part2, norm, b2.reshape(1, -1))


# speedup vs baseline: 13.5608x; 1.1047x over previous
import functools

import jax
import jax.numpy as jnp
from jax import lax
from jax.experimental import pallas as pl
from jax.experimental.pallas import tpu as pltpu
from jax.experimental.pallas import tpu_sc as plsc

N = 10000
E = 320000
NC = 2
NS = 16
NW = NC * NS
EPT = E // NW
CH = 100
NCH = EPT // CH
NP = N
RPT = NP // NS
ZR = 125


def _mesh():
    return plsc.VectorSubcoreMesh(core_axis_name="c", subcore_axis_name="s")


def _fill(ref, nrow, ncol, val):
    v = jnp.full((16,), val, jnp.float32)

    def body(i, carry):
        r = i // (ncol // 16)
        c = i % (ncol // 16)
        ref[r, pl.ds(c * 16, 16)] = v
        return carry

    lax.fori_loop(0, nrow * (ncol // 16), body, 0)


@functools.partial(
    pl.kernel,
    out_type=(
        jax.ShapeDtypeStruct((NC, NP, 16), jnp.float32),
        jax.ShapeDtypeStruct((NC, NP, 16), jnp.float32),
    ),
    mesh=_mesh(),
    compiler_params=pltpu.CompilerParams(use_tc_tiling_on_sc=False),
    scratch_types=[
        pltpu.VMEM((NCH, CH), jnp.int32),
        pltpu.VMEM((NCH, CH), jnp.int32),
        pltpu.VMEM((CH, 16), jnp.float32),
        pltpu.VMEM((ZR, 16), jnp.float32),
        pltpu.VMEM_SHARED((NP, 16), jnp.float32),
        pltpu.VMEM_SHARED((NP, 16), jnp.float32),
        pltpu.SemaphoreType.DMA,
        pltpu.SemaphoreType.DMA,
    ],
)
def _sc_degrees(src_hbm, dst_hbm, do_hbm, di_hbm, isall, idall, ones_v,
                z_v, do_sh, di_sh, semA, semB):
    cid = lax.axis_index("c")
    sid = lax.axis_index("s")
    w = cid * NS + sid

    pltpu.sync_copy(src_hbm.at[w], isall)
    pltpu.sync_copy(dst_hbm.at[w], idall)
    _fill(ones_v, CH, 16, 1.0)
    _fill(z_v, ZR, 16, 0.0)

    def zb(i, carry):
        r0 = sid * RPT + i * ZR
        pltpu.sync_copy(z_v, do_sh.at[pl.ds(r0, ZR)])
        pltpu.sync_copy(z_v, di_sh.at[pl.ds(r0, ZR)])
        return carry

    lax.fori_loop(0, RPT // ZR, zb, 0)
    plsc.subcore_barrier()

    def body(i, carry):
        pltpu.async_copy(ones_v, do_sh.at[isall.at[i]], semA, add=True)
        pltpu.async_copy(ones_v, di_sh.at[idall.at[i]], semB, add=True)

        @pl.when(i > 0)
        def _():
            pltpu.make_async_copy(ones_v, do_sh.at[isall.at[i - 1]],
                                  semA).wait()
            pltpu.make_async_copy(ones_v, di_sh.at[idall.at[i - 1]],
                                  semB).wait()

        return carry

    lax.fori_loop(0, NCH, body, 0)
    pltpu.make_async_copy(ones_v, do_sh.at[isall.at[NCH - 1]], semA).wait()
    pltpu.make_async_copy(ones_v, di_sh.at[idall.at[NCH - 1]], semB).wait()
    plsc.subcore_barrier()

    r0 = sid * RPT
    pltpu.sync_copy(do_sh.at[pl.ds(r0, RPT)], do_hbm.at[cid, pl.ds(r0, RPT)])
    pltpu.sync_copy(di_sh.at[pl.ds(r0, RPT)], di_hbm.at[cid, pl.ds(r0, RPT)])


def _make_sc_agg(D, NBUF, C):
    NCHk = EPT // C
    assert EPT % C == 0 and NCHk % NBUF == 0

    @functools.partial(
        pl.kernel,
        out_type=jax.ShapeDtypeStruct((NC, NP, D), jnp.float32),
        mesh=_mesh(),
        compiler_params=pltpu.CompilerParams(use_tc_tiling_on_sc=False),
        scratch_types=[
            pltpu.VMEM((NCHk, C), jnp.int32),
            pltpu.VMEM((NCHk, C), jnp.int32),
            [pltpu.VMEM((C, D), jnp.float32)] * NBUF,
            [pltpu.SemaphoreType.DMA] * NBUF,
            [pltpu.SemaphoreType.DMA] * NBUF,
            pltpu.VMEM_SHARED((NP, D), jnp.float32),
        ],
    )
    def _sc_agg(y_hbm, src_hbm, dst_hbm, out_hbm, isall, idall, rows,
                semg, sems, agg_sh):
        cid = lax.axis_index("c")
        sid = lax.axis_index("s")
        w = cid * NS + sid

        pltpu.sync_copy(src_hbm.at[w], isall)
        pltpu.sync_copy(dst_hbm.at[w], idall)

        _fill(rows[0], C, D, 0.0)

        def zb(i, carry):
            pltpu.sync_copy(rows[0], agg_sh.at[pl.ds(sid * RPT + i * C, C)])
            return carry

        lax.fori_loop(0, RPT // C, zb, 0)
        if RPT % C:
            pltpu.sync_copy(rows[0].at[pl.ds(0, RPT % C)],
                            agg_sh.at[pl.ds(sid * RPT + (RPT // C) * C,
                                            RPT % C)])
        plsc.subcore_barrier()

        for b in range(NBUF):
            pltpu.async_copy(y_hbm.at[isall.at[b]], rows[b], semg[b])

        def body(g, carry):
            base = g * NBUF
            for b in range(NBUF):
                cc = base + b
                pltpu.make_async_copy(y_hbm.at[isall.at[cc]], rows[b],
                                      semg[b]).wait()
                pltpu.async_copy(rows[b], agg_sh.at[idall.at[cc]], sems[b],
                                 add=True)
                pb = (b - 1) % NBUF
                pcc = cc - 1

                def _drain_refill():
                    pltpu.make_async_copy(rows[pb], agg_sh.at[idall.at[pcc]],
                                          sems[pb]).wait()

                    @pl.when(pcc + NBUF < NCHk)
                    def _():
                        pltpu.async_copy(y_hbm.at[isall.at[pcc + NBUF]],
                                         rows[pb], semg[pb])

                if b == 0:
                    pl.when(g > 0)(_drain_refill)
                else:
                    _drain_refill()

            return carry

        lax.fori_loop(0, NCHk // NBUF, body, 0)
        pltpu.make_async_copy(rows[NBUF - 1], agg_sh.at[idall.at[NCHk - 1]],
                              sems[NBUF - 1]).wait()
        plsc.subcore_barrier()

        r0 = sid * RPT
        pltpu.sync_copy(agg_sh.at[pl.ds(r0, RPT)],
                        out_hbm.at[cid, pl.ds(r0, RPT)])

    return _sc_agg


_sc_agg128 = _make_sc_agg(128, 4, 50)
_sc_agg64 = _make_sc_agg(64, 4, 100)


def _tc_prep_body(do_ref, di_ref, x_ref, norm_ref, y_ref):
    deg_o = do_ref[0] + do_ref[1]
    deg_i = di_ref[0] + di_ref[1]
    ns = lax.rsqrt(jnp.maximum(deg_o, 1.0))
    nd = lax.rsqrt(jnp.maximum(deg_i, 1.0))
    norm_ref[...] = jnp.concatenate([ns, nd], axis=1)
    y_ref[...] = x_ref[...] * ns[:N, 0:1]


def _tc_prep(do_p, di_p, x):
    return pl.pallas_call(
        _tc_prep_body,
        out_shape=(
            jax.ShapeDtypeStruct((NP, 32), jnp.float32),
            jax.ShapeDtypeStruct((N, 128), jnp.float32),
        ),
    )(do_p, di_p, x)


def _tc_layer1_body(p_ref, norm_ref, w1_ref, b1_ref, w2_ref, y2_ref):
    agg = (p_ref[0] + p_ref[1]) * norm_ref[:, 16:17]
    h = jnp.maximum(
        jnp.dot(agg, w1_ref[...], preferred_element_type=jnp.float32)
        + b1_ref[...], 0.0)
    y2_ref[...] = jnp.dot(h, w2_ref[...],
                          preferred_element_type=jnp.float32) * norm_ref[:, 0:1]


def _tc_layer1(part1, norm, W1, b1, W2):
    blk = 1000
    return pl.pallas_call(
        _tc_layer1_body,
        grid=(N // blk,),
        in_specs=[
            pl.BlockSpec((2, blk, 128), lambda i: (0, i, 0)),
            pl.BlockSpec((blk, 32), lambda i: (i, 0)),
            pl.BlockSpec((128, 128), lambda i: (0, 0)),
            pl.BlockSpec((1, 128), lambda i: (0, 0)),
            pl.BlockSpec((128, 64), lambda i: (0, 0)),
        ],
        out_specs=pl.BlockSpec((blk, 64), lambda i: (i, 0)),
        out_shape=jax.ShapeDtypeStruct((N, 64), jnp.float32),
    )(part1, norm, W1, b1, W2)


def _tc_out_body(p_ref, norm_ref, b2_ref, out_ref):
    out_ref[...] = ((p_ref[0] + p_ref[1]) * norm_ref[:, 16:17]
                    + b2_ref[...])


def _tc_out(part2, norm, b2):
    blk = 1000
    return pl.pallas_call(
        _tc_out_body,
        grid=(N // blk,),
        in_specs=[
            pl.BlockSpec((2, blk, 64), lambda i: (0, i, 0)),
            pl.BlockSpec((blk, 32), lambda i: (i, 0)),
            pl.BlockSpec((1, 64), lambda i: (0, 0)),
        ],
        out_specs=pl.BlockSpec((blk, 64), lambda i: (i, 0)),
        out_shape=jax.ShapeDtypeStruct((N, 64), jnp.float32),
    )(part2, norm, b2)


def kernel(x, edge_index, W1, b1, W2, b2):
    src = edge_index[0].reshape(NW, NCH, CH)
    dst = edge_index[1].reshape(NW, NCH, CH)
    src50 = edge_index[0].reshape(NW, EPT // 50, 50)
    dst50 = edge_index[1].reshape(NW, EPT // 50, 50)
    do_p, di_p = _sc_degrees(src, dst)
    norm, y1 = _tc_prep(do_p, di_p, x)
    part1 = _sc_agg128(y1, src50, dst50)
    y2 = _tc_layer1(part1, norm, W1, b1.reshape(1, -1), W2)
    part2 = _sc_agg64(y2, src, dst)
    return _tc_out(part2, norm, b2.reshape(1, -1))


# speedup vs baseline: 13.8969x; 1.0248x over previous
import functools

import jax
import jax.numpy as jnp
from jax import lax
from jax.experimental import pallas as pl
from jax.experimental.pallas import tpu as pltpu
from jax.experimental.pallas import tpu_sc as plsc

N = 10000
E = 320000
NC = 2
NS = 16
NW = NC * NS
EPT = E // NW
CH = 100
NCH = EPT // CH
NP = N
RPT = NP // NS
ZR = 125


def _mesh():
    return plsc.VectorSubcoreMesh(core_axis_name="c", subcore_axis_name="s")


def _fill(ref, nrow, ncol, val):
    v = jnp.full((16,), val, jnp.float32)

    def body(i, carry):
        r = i // (ncol // 16)
        c = i % (ncol // 16)
        ref[r, pl.ds(c * 16, 16)] = v
        return carry

    lax.fori_loop(0, nrow * (ncol // 16), body, 0)


@functools.partial(
    pl.kernel,
    out_type=(
        jax.ShapeDtypeStruct((NC, NP, 16), jnp.float32),
        jax.ShapeDtypeStruct((NC, NP, 16), jnp.float32),
    ),
    mesh=_mesh(),
    compiler_params=pltpu.CompilerParams(use_tc_tiling_on_sc=False),
    scratch_types=[
        pltpu.VMEM((NCH, CH), jnp.int32),
        pltpu.VMEM((NCH, CH), jnp.int32),
        pltpu.VMEM((CH, 16), jnp.float32),
        pltpu.VMEM((ZR, 16), jnp.float32),
        pltpu.VMEM_SHARED((NP, 16), jnp.float32),
        pltpu.VMEM_SHARED((NP, 16), jnp.float32),
        pltpu.SemaphoreType.DMA,
        pltpu.SemaphoreType.DMA,
    ],
)
def _sc_degrees(src_hbm, dst_hbm, do_hbm, di_hbm, isall, idall, ones_v,
                z_v, do_sh, di_sh, semA, semB):
    cid = lax.axis_index("c")
    sid = lax.axis_index("s")
    w = cid * NS + sid

    pltpu.sync_copy(src_hbm.at[w], isall)
    pltpu.sync_copy(dst_hbm.at[w], idall)
    _fill(ones_v, CH, 16, 1.0)
    _fill(z_v, ZR, 16, 0.0)

    def zb(i, carry):
        r0 = sid * RPT + i * ZR
        pltpu.sync_copy(z_v, do_sh.at[pl.ds(r0, ZR)])
        pltpu.sync_copy(z_v, di_sh.at[pl.ds(r0, ZR)])
        return carry

    lax.fori_loop(0, RPT // ZR, zb, 0)
    plsc.subcore_barrier()

    def body(i, carry):
        pltpu.async_copy(ones_v, do_sh.at[isall.at[i]], semA, add=True)
        pltpu.async_copy(ones_v, di_sh.at[idall.at[i]], semB, add=True)

        @pl.when(i > 2)
        def _():
            pltpu.make_async_copy(ones_v, do_sh.at[isall.at[i - 3]],
                                  semA).wait()
            pltpu.make_async_copy(ones_v, di_sh.at[idall.at[i - 3]],
                                  semB).wait()

        return carry

    lax.fori_loop(0, NCH, body, 0)
    for _k in range(3):
        pltpu.make_async_copy(ones_v, do_sh.at[isall.at[NCH - 1]],
                              semA).wait()
        pltpu.make_async_copy(ones_v, di_sh.at[idall.at[NCH - 1]],
                              semB).wait()
    plsc.subcore_barrier()

    r0 = sid * RPT
    pltpu.sync_copy(do_sh.at[pl.ds(r0, RPT)], do_hbm.at[cid, pl.ds(r0, RPT)])
    pltpu.sync_copy(di_sh.at[pl.ds(r0, RPT)], di_hbm.at[cid, pl.ds(r0, RPT)])


def _make_sc_agg(D, NBUF, C):
    NCHk = EPT // C
    assert EPT % C == 0 and NCHk % NBUF == 0

    @functools.partial(
        pl.kernel,
        out_type=jax.ShapeDtypeStruct((NC, NP, D), jnp.float32),
        mesh=_mesh(),
        compiler_params=pltpu.CompilerParams(use_tc_tiling_on_sc=False),
        scratch_types=[
            pltpu.VMEM((NCHk, C), jnp.int32),
            pltpu.VMEM((NCHk, C), jnp.int32),
            [pltpu.VMEM((C, D), jnp.float32)] * NBUF,
            [pltpu.SemaphoreType.DMA] * NBUF,
            [pltpu.SemaphoreType.DMA] * NBUF,
            pltpu.VMEM_SHARED((NP, D), jnp.float32),
        ],
    )
    def _sc_agg(y_hbm, src_hbm, dst_hbm, out_hbm, isall, idall, rows,
                semg, sems, agg_sh):
        cid = lax.axis_index("c")
        sid = lax.axis_index("s")
        w = cid * NS + sid

        pltpu.sync_copy(src_hbm.at[w], isall)
        pltpu.sync_copy(dst_hbm.at[w], idall)

        _fill(rows[0], C, D, 0.0)

        def zb(i, carry):
            pltpu.sync_copy(rows[0], agg_sh.at[pl.ds(sid * RPT + i * C, C)])
            return carry

        lax.fori_loop(0, RPT // C, zb, 0)
        if RPT % C:
            pltpu.sync_copy(rows[0].at[pl.ds(0, RPT % C)],
                            agg_sh.at[pl.ds(sid * RPT + (RPT // C) * C,
                                            RPT % C)])
        plsc.subcore_barrier()

        for b in range(NBUF):
            pltpu.async_copy(y_hbm.at[isall.at[b]], rows[b], semg[b])

        def body(g, carry):
            base = g * NBUF
            for b in range(NBUF):
                cc = base + b
                pltpu.make_async_copy(y_hbm.at[isall.at[cc]], rows[b],
                                      semg[b]).wait()
                pltpu.async_copy(rows[b], agg_sh.at[idall.at[cc]], sems[b],
                                 add=True)
                pb = (b - 1) % NBUF
                pcc = cc - 1

                def _drain_refill():
                    pltpu.make_async_copy(rows[pb], agg_sh.at[idall.at[pcc]],
                                          sems[pb]).wait()

                    @pl.when(pcc + NBUF < NCHk)
                    def _():
                        pltpu.async_copy(y_hbm.at[isall.at[pcc + NBUF]],
                                         rows[pb], semg[pb])

                if b == 0:
                    pl.when(g > 0)(_drain_refill)
                else:
                    _drain_refill()

            return carry

        lax.fori_loop(0, NCHk // NBUF, body, 0)
        pltpu.make_async_copy(rows[NBUF - 1], agg_sh.at[idall.at[NCHk - 1]],
                              sems[NBUF - 1]).wait()
        plsc.subcore_barrier()

        r0 = sid * RPT
        pltpu.sync_copy(agg_sh.at[pl.ds(r0, RPT)],
                        out_hbm.at[cid, pl.ds(r0, RPT)])

    return _sc_agg


_sc_agg128 = _make_sc_agg(128, 4, 50)
_sc_agg64 = _make_sc_agg(64, 8, 50)


def _tc_prep_body(do_ref, di_ref, x_ref, norm_ref, y_ref):
    deg_o = do_ref[0] + do_ref[1]
    deg_i = di_ref[0] + di_ref[1]
    ns = lax.rsqrt(jnp.maximum(deg_o, 1.0))
    nd = lax.rsqrt(jnp.maximum(deg_i, 1.0))
    norm_ref[...] = jnp.concatenate([ns, nd], axis=1)
    y_ref[...] = x_ref[...] * ns[:N, 0:1]


def _tc_prep(do_p, di_p, x):
    return pl.pallas_call(
        _tc_prep_body,
        out_shape=(
            jax.ShapeDtypeStruct((NP, 32), jnp.float32),
            jax.ShapeDtypeStruct((N, 128), jnp.float32),
        ),
    )(do_p, di_p, x)


def _tc_layer1_body(p_ref, norm_ref, w1_ref, b1_ref, w2_ref, y2_ref):
    agg = (p_ref[0] + p_ref[1]) * norm_ref[:, 16:17]
    h = jnp.maximum(
        jnp.dot(agg, w1_ref[...], preferred_element_type=jnp.float32)
        + b1_ref[...], 0.0)
    y2_ref[...] = jnp.dot(h, w2_ref[...],
                          preferred_element_type=jnp.float32) * norm_ref[:, 0:1]


def _tc_layer1(part1, norm, W1, b1, W2):
    blk = 1000
    return pl.pallas_call(
        _tc_layer1_body,
        grid=(N // blk,),
        in_specs=[
            pl.BlockSpec((2, blk, 128), lambda i: (0, i, 0)),
            pl.BlockSpec((blk, 32), lambda i: (i, 0)),
            pl.BlockSpec((128, 128), lambda i: (0, 0)),
            pl.BlockSpec((1, 128), lambda i: (0, 0)),
            pl.BlockSpec((128, 64), lambda i: (0, 0)),
        ],
        out_specs=pl.BlockSpec((blk, 64), lambda i: (i, 0)),
        out_shape=jax.ShapeDtypeStruct((N, 64), jnp.float32),
    )(part1, norm, W1, b1, W2)


def _tc_out_body(p_ref, norm_ref, b2_ref, out_ref):
    out_ref[...] = ((p_ref[0] + p_ref[1]) * norm_ref[:, 16:17]
                    + b2_ref[...])


def _tc_out(part2, norm, b2):
    blk = 1000
    return pl.pallas_call(
        _tc_out_body,
        grid=(N // blk,),
        in_specs=[
            pl.BlockSpec((2, blk, 64), lambda i: (0, i, 0)),
            pl.BlockSpec((blk, 32), lambda i: (i, 0)),
            pl.BlockSpec((1, 64), lambda i: (0, 0)),
        ],
        out_specs=pl.BlockSpec((blk, 64), lambda i: (i, 0)),
        out_shape=jax.ShapeDtypeStruct((N, 64), jnp.float32),
    )(part2, norm, b2)


def kernel(x, edge_index, W1, b1, W2, b2):
    src = edge_index[0].reshape(NW, NCH, CH)
    dst = edge_index[1].reshape(NW, NCH, CH)
    src50 = edge_index[0].reshape(NW, EPT // 50, 50)
    dst50 = edge_index[1].reshape(NW, EPT // 50, 50)
    do_p, di_p = _sc_degrees(src, dst)
    norm, y1 = _tc_prep(do_p, di_p, x)
    part1 = _sc_agg128(y1, src50, dst50)
    y2 = _tc_layer1(part1, norm, W1, b1.reshape(1, -1), W2)
    part2 = _sc_agg64(y2, src50, dst50)
    return _tc_out(part2, norm, b2.reshape(1, -1))
